# baseline (speedup 1.0000x reference)
import functools

import jax
import jax.numpy as jnp
from jax import lax
from jax.experimental import pallas as pl
from jax.experimental.pallas import tpu as pltpu
from jax.experimental.pallas import tpu_sc as plsc

_B, _S, _D = 4, 4096, 768
_NC, _NS = 2, 16
_NW = _NC * _NS
_R = (_B * _S) // _NW
_CHUNK = 64
_NCH = _R // _CHUNK

_mesh = plsc.VectorSubcoreMesh(core_axis_name="c", subcore_axis_name="s")


@functools.partial(
    pl.kernel,
    out_type=jax.ShapeDtypeStruct((_B * _S, _D), jnp.float32),
    mesh=_mesh,
    scratch_types=[
        pltpu.VMEM((_NCH, _CHUNK), jnp.int32),
        pltpu.VMEM((_CHUNK, _D), jnp.float32),
        pltpu.VMEM((_CHUNK, _D), jnp.float32),
        pltpu.SemaphoreType.DMA,
        pltpu.SemaphoreType.DMA,
    ],
)
def _emb_kernel(x_hbm, tok_hbm, pos_hbm, out_hbm, idx_v, buf_a, buf_b, sem_a, sem_b):
    cid = lax.axis_index("c")
    sid = lax.axis_index("s")
    wid = sid * _NC + cid
    base = wid * _R
    s_off = lax.rem(base, _S)
    pltpu.sync_copy(x_hbm.at[wid], idx_v)
    for c in range(_NCH):
        ga = pltpu.async_copy(tok_hbm.at[idx_v.at[c]], buf_a, sem_a)
        gb = pltpu.async_copy(
            pos_hbm.at[pl.ds(s_off + c * _CHUNK, _CHUNK)], buf_b, sem_b)
        ga.wait()
        gb.wait()

        def _row_add(r, carry):
            for i in range(_D // 16):
                sl = pl.ds(i * 16, 16)
                buf_a[r, sl] = buf_a[r, sl] + buf_b[r, sl]
            return carry

        lax.fori_loop(0, _CHUNK, _row_add, 0)
        pltpu.sync_copy(buf_a, out_hbm.at[pl.ds(base + c * _CHUNK, _CHUNK)])


def kernel(x, token_table, pos_table):
    x3 = x.reshape(_NW, _NCH, _CHUNK).astype(jnp.int32)
    out = _emb_kernel(x3, token_table, pos_table)
    return out.reshape(_B, _S, _D)


# speedup vs baseline: 1.4067x; 1.4067x over previous
import functools

import jax
import jax.numpy as jnp
from jax import lax
from jax.experimental import pallas as pl
from jax.experimental.pallas import tpu as pltpu
from jax.experimental.pallas import tpu_sc as plsc

_B, _S, _D = 4, 4096, 768
_NC, _NS = 2, 16
_NW = _NC * _NS
_SW = _S // _NW
_CHUNK = 32
_NG = _SW // _CHUNK
_NCHUNKS = _NG * _B

_mesh = plsc.VectorSubcoreMesh(core_axis_name="c", subcore_axis_name="s")


@functools.partial(
    pl.kernel,
    out_type=jax.ShapeDtypeStruct((_B * _S, _D), jnp.float32),
    mesh=_mesh,
    scratch_types=[
        pltpu.VMEM((_B, _SW), jnp.int32),
        pltpu.VMEM((_CHUNK, _D), jnp.float32),
        pltpu.VMEM((_CHUNK, _D), jnp.float32),
        pltpu.VMEM((_CHUNK, _D), jnp.float32),
        pltpu.VMEM((_CHUNK, _D), jnp.float32),
        pltpu.SemaphoreType.DMA,
        pltpu.SemaphoreType.DMA,
        pltpu.SemaphoreType.DMA,
        pltpu.SemaphoreType.DMA,
        pltpu.SemaphoreType.DMA,
        pltpu.SemaphoreType.DMA,
    ],
)
def _emb_kernel(x_hbm, tok_hbm, pos_hbm, out_hbm, idx_v,
                tbuf0, tbuf1, pbuf0, pbuf1,
                sg0, sg1, sp0, sp1, ss0, ss1):
    cid = lax.axis_index("c")
    sid = lax.axis_index("s")
    wid = sid * _NC + cid
    s_base = wid * _SW

    tb = [tbuf0, tbuf1]
    pb = [pbuf0, pbuf1]
    sg = [sg0, sg1]
    sp = [sp0, sp1]
    ss = [ss0, ss1]

    pltpu.sync_copy(x_hbm.at[:, pl.ds(s_base, _SW)], idx_v)

    def _gather(i, buf, sem):
        k, b = divmod(i, _B)
        return pltpu.async_copy(
            tok_hbm.at[idx_v.at[b, pl.ds(k * _CHUNK, _CHUNK)]], buf, sem)

    def _pos_load(k, buf, sem):
        return pltpu.async_copy(
            pos_hbm.at[pl.ds(s_base + k * _CHUNK, _CHUNK)], buf, sem)

    gather_d = [None, None]
    pos_d = [None, None]
    store_d = [None, None]

    pos_d[0] = _pos_load(0, pb[0], sp[0])
    gather_d[0] = _gather(0, tb[0], sg[0])

    for i in range(_NCHUNKS):
        k, b = divmod(i, _B)
        cur = i % 2
        nxt = 1 - cur
        if i + 1 < _NCHUNKS:
            if store_d[nxt] is not None:
                store_d[nxt].wait()
                store_d[nxt] = None
            gather_d[nxt] = _gather(i + 1, tb[nxt], sg[nxt])
        if b == 0:
            if k + 1 < _NG:
                pos_d[(k + 1) % 2] = _pos_load(k + 1, pb[(k + 1) % 2],
                                               sp[(k + 1) % 2])
            pos_d[k % 2].wait()
        gather_d[cur].wait()

        tcur = tb[cur]
        pcur = pb[k % 2]

        def _row_add(r, carry):
            for j in range(_D // 16):
                sl = pl.ds(j * 16, 16)
                plsc.addupdate(tcur.at[r, sl], pcur[r, sl])
            return carry

        lax.fori_loop(0, _CHUNK, _row_add, 0)

        if store_d[cur] is not None:
            store_d[cur].wait()
            store_d[cur] = None
        store_d[cur] = pltpu.async_copy(
            tcur, out_hbm.at[pl.ds(b * _S + s_base + k * _CHUNK, _CHUNK)],
            ss[cur])

    for j in range(2):
        if store_d[j] is not None:
            store_d[j].wait()


def kernel(x, token_table, pos_table):
    out = _emb_kernel(x.astype(jnp.int32), token_table, pos_table)
    return out.reshape(_B, _S, _D)
